# baseline (speedup 1.0000x reference)
import functools

import jax
import jax.numpy as jnp
from jax import lax
from jax.experimental import pallas as pl
from jax.experimental.pallas import tpu as pltpu
from jax.experimental.pallas import tpu_sc as plsc

N = 10000
D = 128
E = 320000

NC = 2
NS = 16
L = 16

CHUNK = 256
EPT = 10240
EPAD = EPT * NC * NS
EROWS = EPAD // 128
ROWS_PER_TILE = EROWS // (NC * NS)
ROWS_PER_TILE_1SC = EROWS // NS
NACC = 10240
RPT = NACC // NS

_mesh = plsc.VectorSubcoreMesh(core_axis_name="c", subcore_axis_name="s")


def _sc_degree_body(dst1d, deg_out, hist_sh, idx_v, ones_v, hbuf, sem):
    c = lax.axis_index("c")
    s = lax.axis_index("s")

    @pl.when(c == 0)
    def _():
        def zrow(i, _):
            hbuf[pl.ds(i * L, L)] = jnp.zeros((L,), jnp.float32)
            return 0
        lax.fori_loop(0, RPT // L, zrow, 0)
        pltpu.sync_copy(hbuf, hist_sh.at[pl.ds(s * RPT, RPT)])

        def orow(i, _):
            ones_v[pl.ds(i * L, L)] = jnp.ones((L,), jnp.float32)
            return 0
        lax.fori_loop(0, 128 // L, orow, 0)
        plsc.subcore_barrier()

        base = s * ROWS_PER_TILE_1SC * 128

        def srow(g, _):
            pltpu.sync_copy(dst1d.at[pl.ds(base + g * 128, 128)], idx_v)
            pltpu.sync_copy(ones_v, hist_sh.at[idx_v], add=True)
            return 0
        lax.fori_loop(0, ROWS_PER_TILE_1SC, srow, 0)
        plsc.subcore_barrier()

        pltpu.sync_copy(hist_sh.at[pl.ds(s * RPT, RPT)], hbuf)
        pltpu.sync_copy(hbuf, deg_out.at[pl.ds(s * RPT, RPT)])


_sc_degree = functools.partial(
    pl.kernel,
    out_type=jax.ShapeDtypeStruct((NACC,), jnp.float32),
    mesh=_mesh,
    scratch_types=[
        pltpu.VMEM_SHARED((NACC,), jnp.float32),
        pltpu.VMEM((128,), jnp.int32),
        pltpu.VMEM((128,), jnp.float32),
        pltpu.VMEM((RPT,), jnp.float32),
        pltpu.SemaphoreType.DMA,
    ],
)(_sc_degree_body)


_CH = 64
_NSL = 4
_EA = 4608
_EB = 2 * EPT - _EA


def _sc_aggregate_body(hs, src1d, dst1d, acc_out,
                       acc_sh, src_all,
                       dst_v0, dst_v1, dst_v2, dst_v3,
                       rows0, rows1, rows2, rows3,
                       isem0, isem1, isem2, isem3,
                       gsem0, gsem1, gsem2, gsem3,
                       ssem0, ssem1, ssem2, ssem3):
    c = lax.axis_index("c")
    s = lax.axis_index("s")
    dst_vs = (dst_v0, dst_v1, dst_v2, dst_v3)
    rows_vs = (rows0, rows1, rows2, rows3)
    isems = (isem0, isem1, isem2, isem3)
    gsems = (gsem0, gsem1, gsem2, gsem3)
    ssems = (ssem0, ssem1, ssem2, ssem3)
    hop = _CH

    def zrow(i, _):
        for k in range(8):
            rows0[i, pl.ds(k * L, L)] = jnp.zeros((L,), jnp.float32)
        return 0
    lax.fori_loop(0, hop, zrow, 0)
    zcps = [
        pltpu.make_async_copy(rows0, acc_sh.at[pl.ds(s * RPT + p * hop, hop)],
                              gsem0)
        for p in range(RPT // hop)
    ]
    for cp in zcps:
        cp.start()
    for cp in zcps:
        cp.wait()
    plsc.subcore_barrier()

    base = jnp.where(c == 0, s * _EA, NS * _EA + s * _EB)
    nr = jnp.where(c == 0, _EA // (_CH * _NSL), _EB // (_CH * _NSL))
    pltpu.sync_copy(src1d.at[pl.ds(base, _EB)], src_all)

    def start_slot(g, j):
        pltpu.async_copy(dst1d.at[pl.ds(base + g * _CH, _CH)], dst_vs[j],
                         isems[j])
        pltpu.async_copy(hs.at[src_all.at[pl.ds(g * _CH, _CH)]], rows_vs[j],
                         gsems[j])

    for j in range(_NSL):
        start_slot(j, j)

    def fire_scatter(g, j):
        pltpu.make_async_copy(dst1d.at[pl.ds(base + g * _CH, _CH)], dst_vs[j],
                              isems[j]).wait()
        pltpu.make_async_copy(hs.at[src_all.at[pl.ds(g * _CH, _CH)]],
                              rows_vs[j], gsems[j]).wait()
        pltpu.make_async_copy(rows_vs[j], acc_sh.at[dst_vs[j]],
                              ssems[j]).start(add=True)

    def round_(r, _):
        g0 = r * _NSL
        for j in range(_NSL):
            fire_scatter(g0 + j, j)

        @pl.when(r < nr - 1)
        def _():
            for j in range(_NSL):
                pltpu.make_async_copy(rows_vs[j], acc_sh.at[dst_vs[j]],
                                      ssems[j]).wait()
                start_slot(g0 + _NSL + j, j)
        return 0

    lax.fori_loop(0, nr, round_, 0)
    for j in range(_NSL):
        pltpu.make_async_copy(rows_vs[j], acc_sh.at[dst_vs[j]],
                              ssems[j]).wait()
    plsc.subcore_barrier()

    nhop = RPT // hop
    for p in range(nhop):
        j = p % _NSL
        if p >= _NSL:
            pltpu.make_async_copy(
                rows_vs[j],
                acc_out.at[c, pl.ds(s * RPT + (p - _NSL) * hop, hop)],
                gsems[j]).wait()
        pltpu.sync_copy(acc_sh.at[pl.ds(s * RPT + p * hop, hop)], rows_vs[j])
        pltpu.make_async_copy(
            rows_vs[j], acc_out.at[c, pl.ds(s * RPT + p * hop, hop)],
            gsems[j]).start()
    for p in range(max(0, nhop - _NSL), nhop):
        j = p % _NSL
        pltpu.make_async_copy(
            rows_vs[j], acc_out.at[c, pl.ds(s * RPT + p * hop, hop)],
            gsems[j]).wait()


_sc_aggregate = functools.partial(
    pl.kernel,
    out_type=jax.ShapeDtypeStruct((NC, NACC, D), jnp.float32),
    mesh=_mesh,
    scratch_types=[
        pltpu.VMEM_SHARED((NACC, D), jnp.float32),
        pltpu.VMEM((_EB,), jnp.int32),
        pltpu.VMEM((_CH,), jnp.int32),
        pltpu.VMEM((_CH,), jnp.int32),
        pltpu.VMEM((_CH,), jnp.int32),
        pltpu.VMEM((_CH,), jnp.int32),
        pltpu.VMEM((_CH, D), jnp.float32),
        pltpu.VMEM((_CH, D), jnp.float32),
        pltpu.VMEM((_CH, D), jnp.float32),
        pltpu.VMEM((_CH, D), jnp.float32),
    ] + [pltpu.SemaphoreType.DMA] * 12,
)(_sc_aggregate_body)


_BLK = 1000
_GRID = N // _BLK


def _tc_matmul_body(x_ref, w_ref, h_ref):
    h_ref[...] = jnp.dot(x_ref[...], w_ref[...],
                         preferred_element_type=jnp.float32)


def _tc_matmul(x, w):
    return pl.pallas_call(
        _tc_matmul_body,
        grid=(_GRID,),
        in_specs=[
            pl.BlockSpec((_BLK, D), lambda i: (i, 0)),
            pl.BlockSpec((D, D), lambda i: (0, 0)),
        ],
        out_specs=pl.BlockSpec((_BLK, D), lambda i: (i, 0)),
        out_shape=jax.ShapeDtypeStruct((N, D), jnp.float32),
    )(x, w)


def _tc_scale_body(deg_ref, h_ref, hs_ref):
    dinv = lax.rsqrt(deg_ref[...] + 1.0)
    hs_ref[...] = h_ref[...] * dinv


def _tc_scale(deg2d, h):
    return pl.pallas_call(
        _tc_scale_body,
        grid=(_GRID,),
        in_specs=[
            pl.BlockSpec((_BLK, 1), lambda i: (i, 0)),
            pl.BlockSpec((_BLK, D), lambda i: (i, 0)),
        ],
        out_specs=pl.BlockSpec((_BLK, D), lambda i: (i, 0)),
        out_shape=jax.ShapeDtypeStruct((N, D), jnp.float32),
    )(deg2d, h)


def _tc_mid_body(deg_ref, aa_ref, ab_ref, hs_ref, b_ref, w_ref, out_ref):
    dinv = lax.rsqrt(deg_ref[...] + 1.0)
    agg = aa_ref[0] + ab_ref[0] + hs_ref[...]
    h1 = jnp.maximum(agg * dinv + b_ref[...], 0.0)
    h = jnp.dot(h1, w_ref[...], preferred_element_type=jnp.float32)
    out_ref[...] = h * dinv


def _tc_mid(deg2d, acc, hs, b2d, w):
    return pl.pallas_call(
        _tc_mid_body,
        grid=(_GRID,),
        in_specs=[
            pl.BlockSpec((_BLK, 1), lambda i: (i, 0)),
            pl.BlockSpec((1, _BLK, D), lambda i: (0, i, 0)),
            pl.BlockSpec((1, _BLK, D), lambda i: (1, i, 0)),
            pl.BlockSpec((_BLK, D), lambda i: (i, 0)),
            pl.BlockSpec((1, D), lambda i: (0, 0)),
            pl.BlockSpec((D, D), lambda i: (0, 0)),
        ],
        out_specs=pl.BlockSpec((_BLK, D), lambda i: (i, 0)),
        out_shape=jax.ShapeDtypeStruct((N, D), jnp.float32),
    )(deg2d, acc, acc, hs, b2d, w)


def _tc_final_body(deg_ref, aa_ref, ab_ref, hs_ref, b_ref, x_ref, out_ref):
    dinv = lax.rsqrt(deg_ref[...] + 1.0)
    agg = aa_ref[0] + ab_ref[0] + hs_ref[...]
    h2 = jnp.maximum(agg * dinv + b_ref[...], 0.0)
    out_ref[...] = (x_ref[...] + h2) * 0.5


def _tc_final(deg2d, acc, hs, b2d, x):
    return pl.pallas_call(
        _tc_final_body,
        grid=(_GRID,),
        in_specs=[
            pl.BlockSpec((_BLK, 1), lambda i: (i, 0)),
            pl.BlockSpec((1, _BLK, D), lambda i: (0, i, 0)),
            pl.BlockSpec((1, _BLK, D), lambda i: (1, i, 0)),
            pl.BlockSpec((_BLK, D), lambda i: (i, 0)),
            pl.BlockSpec((1, D), lambda i: (0, 0)),
            pl.BlockSpec((_BLK, D), lambda i: (i, 0)),
        ],
        out_specs=pl.BlockSpec((_BLK, D), lambda i: (i, 0)),
        out_shape=jax.ShapeDtypeStruct((N, D), jnp.float32),
    )(deg2d, acc, acc, hs, b2d, x)


def kernel(x, edge_index, W1, b1, W2, b2):
    src = edge_index[0]
    dst = edge_index[1]
    pad = EPAD - E
    src1d = jnp.concatenate([src, jnp.zeros((pad,), jnp.int32)])
    dst1d = jnp.concatenate([dst, jnp.full((pad,), N, jnp.int32)])

    h1raw = _tc_matmul(x, W1)
    deg2d = _sc_degree(dst1d).reshape(NACC, 1)

    hs1 = _tc_scale(deg2d, h1raw)
    acc1 = _sc_aggregate(hs1, src1d, dst1d)
    hs2 = _tc_mid(deg2d, acc1, hs1, b1.reshape(1, D), W2)
    acc2 = _sc_aggregate(hs2, src1d, dst1d)
    return _tc_final(deg2d, acc2, hs2, b2.reshape(1, D), x)


# speedup vs baseline: 1.0529x; 1.0529x over previous
import functools

import jax
import jax.numpy as jnp
from jax import lax
from jax.experimental import pallas as pl
from jax.experimental.pallas import tpu as pltpu
from jax.experimental.pallas import tpu_sc as plsc

N = 10000
D = 128
E = 320000

NC = 2
NS = 16
L = 16

CHUNK = 256
EPT = 10240
EPAD = EPT * NC * NS
EROWS = EPAD // 128
ROWS_PER_TILE = EROWS // (NC * NS)
ROWS_PER_TILE_1SC = EROWS // NS
NACC = 10240
RPT = NACC // NS

_mesh = plsc.VectorSubcoreMesh(core_axis_name="c", subcore_axis_name="s")


def _sc_degree_body(dst1d, deg_out, hist_sh, idx_v, ones_v, hbuf, sem):
    c = lax.axis_index("c")
    s = lax.axis_index("s")

    @pl.when(c == 0)
    def _():
        def zrow(i, _):
            hbuf[pl.ds(i * L, L)] = jnp.zeros((L,), jnp.float32)
            return 0
        lax.fori_loop(0, RPT // L, zrow, 0)
        pltpu.sync_copy(hbuf, hist_sh.at[pl.ds(s * RPT, RPT)])

        def orow(i, _):
            ones_v[pl.ds(i * L, L)] = jnp.ones((L,), jnp.float32)
            return 0
        lax.fori_loop(0, 128 // L, orow, 0)
        plsc.subcore_barrier()

        base = s * ROWS_PER_TILE_1SC * 128

        def srow(g, _):
            pltpu.sync_copy(dst1d.at[pl.ds(base + g * 128, 128)], idx_v)
            pltpu.sync_copy(ones_v, hist_sh.at[idx_v], add=True)
            return 0
        lax.fori_loop(0, ROWS_PER_TILE_1SC, srow, 0)
        plsc.subcore_barrier()

        pltpu.sync_copy(hist_sh.at[pl.ds(s * RPT, RPT)], hbuf)
        pltpu.sync_copy(hbuf, deg_out.at[pl.ds(s * RPT, RPT)])


_sc_degree = functools.partial(
    pl.kernel,
    out_type=jax.ShapeDtypeStruct((NACC,), jnp.float32),
    mesh=_mesh,
    scratch_types=[
        pltpu.VMEM_SHARED((NACC,), jnp.float32),
        pltpu.VMEM((128,), jnp.int32),
        pltpu.VMEM((128,), jnp.float32),
        pltpu.VMEM((RPT,), jnp.float32),
        pltpu.SemaphoreType.DMA,
    ],
)(_sc_degree_body)


_CH = 64
_NSL = 4
_EA = EPT
_EB = 2 * EPT - _EA


def _sc_aggregate_body(hs, src1d, dst1d, acc_out,
                       acc_sh, src_all,
                       dst_v0, dst_v1, dst_v2, dst_v3,
                       rows0, rows1, rows2, rows3,
                       isem0, isem1, isem2, isem3,
                       gsem0, gsem1, gsem2, gsem3,
                       ssem0, ssem1, ssem2, ssem3):
    c = lax.axis_index("c")
    s = lax.axis_index("s")
    dst_vs = (dst_v0, dst_v1, dst_v2, dst_v3)
    rows_vs = (rows0, rows1, rows2, rows3)
    isems = (isem0, isem1, isem2, isem3)
    gsems = (gsem0, gsem1, gsem2, gsem3)
    ssems = (ssem0, ssem1, ssem2, ssem3)
    hop = _CH

    def zrow(i, _):
        for k in range(8):
            rows0[i, pl.ds(k * L, L)] = jnp.zeros((L,), jnp.float32)
        return 0
    lax.fori_loop(0, hop, zrow, 0)
    zcps = [
        pltpu.make_async_copy(rows0, acc_sh.at[pl.ds(s * RPT + p * hop, hop)],
                              gsem0)
        for p in range(RPT // hop)
    ]
    for cp in zcps:
        cp.start()
    for cp in zcps:
        cp.wait()
    plsc.subcore_barrier()

    base = jnp.where(c == 0, s * _EA, NS * _EA + s * _EB)
    nr = jnp.where(c == 0, _EA // (_CH * _NSL), _EB // (_CH * _NSL))
    pltpu.sync_copy(src1d.at[pl.ds(base, _EB)], src_all)

    def start_slot(g, j):
        pltpu.async_copy(dst1d.at[pl.ds(base + g * _CH, _CH)], dst_vs[j],
                         isems[j])
        pltpu.async_copy(hs.at[src_all.at[pl.ds(g * _CH, _CH)]], rows_vs[j],
                         gsems[j])

    for j in range(_NSL):
        start_slot(j, j)

    def fire_scatter(g, j):
        pltpu.make_async_copy(dst1d.at[pl.ds(base + g * _CH, _CH)], dst_vs[j],
                              isems[j]).wait()
        pltpu.make_async_copy(hs.at[src_all.at[pl.ds(g * _CH, _CH)]],
                              rows_vs[j], gsems[j]).wait()
        pltpu.make_async_copy(rows_vs[j], acc_sh.at[dst_vs[j]],
                              ssems[j]).start(add=True)

    def round_(r, _):
        g0 = r * _NSL
        for j in range(_NSL):
            fire_scatter(g0 + j, j)

        @pl.when(r < nr - 1)
        def _():
            for j in range(_NSL):
                pltpu.make_async_copy(rows_vs[j], acc_sh.at[dst_vs[j]],
                                      ssems[j]).wait()
                start_slot(g0 + _NSL + j, j)
        return 0

    lax.fori_loop(0, nr, round_, 0)
    for j in range(_NSL):
        pltpu.make_async_copy(rows_vs[j], acc_sh.at[dst_vs[j]],
                              ssems[j]).wait()
    plsc.subcore_barrier()

    nhop = RPT // hop
    for p in range(nhop):
        j = p % _NSL
        if p >= _NSL:
            pltpu.make_async_copy(
                rows_vs[j],
                acc_out.at[c, pl.ds(s * RPT + (p - _NSL) * hop, hop)],
                gsems[j]).wait()
        pltpu.sync_copy(acc_sh.at[pl.ds(s * RPT + p * hop, hop)], rows_vs[j])
        pltpu.make_async_copy(
            rows_vs[j], acc_out.at[c, pl.ds(s * RPT + p * hop, hop)],
            gsems[j]).start()
    for p in range(max(0, nhop - _NSL), nhop):
        j = p % _NSL
        pltpu.make_async_copy(
            rows_vs[j], acc_out.at[c, pl.ds(s * RPT + p * hop, hop)],
            gsems[j]).wait()


_sc_aggregate = functools.partial(
    pl.kernel,
    out_type=jax.ShapeDtypeStruct((NC, NACC, D), jnp.float32),
    mesh=_mesh,
    scratch_types=[
        pltpu.VMEM_SHARED((NACC, D), jnp.float32),
        pltpu.VMEM((_EB,), jnp.int32),
        pltpu.VMEM((_CH,), jnp.int32),
        pltpu.VMEM((_CH,), jnp.int32),
        pltpu.VMEM((_CH,), jnp.int32),
        pltpu.VMEM((_CH,), jnp.int32),
        pltpu.VMEM((_CH, D), jnp.float32),
        pltpu.VMEM((_CH, D), jnp.float32),
        pltpu.VMEM((_CH, D), jnp.float32),
        pltpu.VMEM((_CH, D), jnp.float32),
    ] + [pltpu.SemaphoreType.DMA] * 12,
)(_sc_aggregate_body)


_BLK = 1000
_GRID = N // _BLK


def _tc_matmul_body(x_ref, w_ref, h_ref):
    h_ref[...] = jnp.dot(x_ref[...], w_ref[...],
                         preferred_element_type=jnp.float32)


def _tc_matmul(x, w):
    return pl.pallas_call(
        _tc_matmul_body,
        grid=(_GRID,),
        in_specs=[
            pl.BlockSpec((_BLK, D), lambda i: (i, 0)),
            pl.BlockSpec((D, D), lambda i: (0, 0)),
        ],
        out_specs=pl.BlockSpec((_BLK, D), lambda i: (i, 0)),
        out_shape=jax.ShapeDtypeStruct((N, D), jnp.float32),
    )(x, w)


def _tc_scale_body(deg_ref, h_ref, hs_ref):
    dinv = lax.rsqrt(deg_ref[...] + 1.0)
    hs_ref[...] = h_ref[...] * dinv


def _tc_scale(deg2d, h):
    return pl.pallas_call(
        _tc_scale_body,
        grid=(_GRID,),
        in_specs=[
            pl.BlockSpec((_BLK, 1), lambda i: (i, 0)),
            pl.BlockSpec((_BLK, D), lambda i: (i, 0)),
        ],
        out_specs=pl.BlockSpec((_BLK, D), lambda i: (i, 0)),
        out_shape=jax.ShapeDtypeStruct((N, D), jnp.float32),
    )(deg2d, h)


def _tc_mid_body(deg_ref, aa_ref, ab_ref, hs_ref, b_ref, w_ref, out_ref):
    dinv = lax.rsqrt(deg_ref[...] + 1.0)
    agg = aa_ref[0] + ab_ref[0] + hs_ref[...]
    h1 = jnp.maximum(agg * dinv + b_ref[...], 0.0)
    h = jnp.dot(h1, w_ref[...], preferred_element_type=jnp.float32)
    out_ref[...] = h * dinv


def _tc_mid(deg2d, acc, hs, b2d, w):
    return pl.pallas_call(
        _tc_mid_body,
        grid=(_GRID,),
        in_specs=[
            pl.BlockSpec((_BLK, 1), lambda i: (i, 0)),
            pl.BlockSpec((1, _BLK, D), lambda i: (0, i, 0)),
            pl.BlockSpec((1, _BLK, D), lambda i: (1, i, 0)),
            pl.BlockSpec((_BLK, D), lambda i: (i, 0)),
            pl.BlockSpec((1, D), lambda i: (0, 0)),
            pl.BlockSpec((D, D), lambda i: (0, 0)),
        ],
        out_specs=pl.BlockSpec((_BLK, D), lambda i: (i, 0)),
        out_shape=jax.ShapeDtypeStruct((N, D), jnp.float32),
    )(deg2d, acc, acc, hs, b2d, w)


def _tc_final_body(deg_ref, aa_ref, ab_ref, hs_ref, b_ref, x_ref, out_ref):
    dinv = lax.rsqrt(deg_ref[...] + 1.0)
    agg = aa_ref[0] + ab_ref[0] + hs_ref[...]
    h2 = jnp.maximum(agg * dinv + b_ref[...], 0.0)
    out_ref[...] = (x_ref[...] + h2) * 0.5


def _tc_final(deg2d, acc, hs, b2d, x):
    return pl.pallas_call(
        _tc_final_body,
        grid=(_GRID,),
        in_specs=[
            pl.BlockSpec((_BLK, 1), lambda i: (i, 0)),
            pl.BlockSpec((1, _BLK, D), lambda i: (0, i, 0)),
            pl.BlockSpec((1, _BLK, D), lambda i: (1, i, 0)),
            pl.BlockSpec((_BLK, D), lambda i: (i, 0)),
            pl.BlockSpec((1, D), lambda i: (0, 0)),
            pl.BlockSpec((_BLK, D), lambda i: (i, 0)),
        ],
        out_specs=pl.BlockSpec((_BLK, D), lambda i: (i, 0)),
        out_shape=jax.ShapeDtypeStruct((N, D), jnp.float32),
    )(deg2d, acc, acc, hs, b2d, x)


def kernel(x, edge_index, W1, b1, W2, b2):
    src = edge_index[0]
    dst = edge_index[1]
    pad = EPAD - E
    src1d = jnp.concatenate([src, jnp.zeros((pad,), jnp.int32)])
    dst1d = jnp.concatenate([dst, jnp.full((pad,), N, jnp.int32)])

    h1raw = _tc_matmul(x, W1)
    deg2d = _sc_degree(dst1d).reshape(NACC, 1)

    hs1 = _tc_scale(deg2d, h1raw)
    acc1 = _sc_aggregate(hs1, src1d, dst1d)
    hs2 = _tc_mid(deg2d, acc1, hs1, b1.reshape(1, D), W2)
    acc2 = _sc_aggregate(hs2, src1d, dst1d)
    return _tc_final(deg2d, acc2, hs2, b2.reshape(1, D), x)
